# baseline (speedup 1.0000x reference)
import functools
import jax
import jax.numpy as jnp
from jax import lax
from jax.experimental import pallas as pl
from jax.experimental.pallas import tpu as pltpu
from jax.experimental.pallas import tpu_sc as plsc

RES = 32
V = RES ** 3
B = 4
N = 65536
C = 64

NC = 2
NS = 16
CH = C // NC
PT = N // NS
VT = V // NS
RPT = PT // 128
NCHUNK = 16
INVN = 1.0 / N

_MESH = plsc.VectorSubcoreMesh(
    core_axis_name="c", subcore_axis_name="s", num_cores=NC, num_subcores=NS)


def _idx_body(ct_ref, idx_ref):
    c3 = ct_ref[0]
    mean = jnp.mean(c3, axis=1, keepdims=True)
    cc = c3 - mean
    n2 = jnp.sum(cc * cc, axis=0, keepdims=True)
    m = jnp.sqrt(jnp.max(n2))
    cn = cc / (m * 2.0) + 0.5
    v = jnp.clip(cn * float(RES), 0.0, float(RES - 1))
    vi = jnp.round(v).astype(jnp.int32)
    idx_ref[0] = vi[0:1] * (RES * RES) + vi[1:2] * RES + vi[2:3]


def _idx_tc(ct):
    return pl.pallas_call(
        _idx_body,
        grid=(B,),
        in_specs=[pl.BlockSpec((1, 3, N), lambda b: (b, 0, 0))],
        out_specs=pl.BlockSpec((1, 1, N), lambda b: (b, 0, 0)),
        out_shape=jax.ShapeDtypeStruct((B, 1, N), jnp.int32),
    )(ct)


@functools.partial(
    pl.kernel,
    out_type=[
        jax.ShapeDtypeStruct((3, 512, 4, 128), jnp.float32),
        jax.ShapeDtypeStruct((B, V // 2, 128), jnp.float32),
    ],
    mesh=_MESH,
    compiler_params=pltpu.CompilerParams(
        use_tc_tiling_on_sc=False, needs_layout_passes=False),
    scratch_types=[
        pltpu.VMEM((RPT, 1, 128), jnp.float32),
        pltpu.VMEM((RPT, 1, 128), jnp.float32),
        pltpu.VMEM((RPT, 1, 128), jnp.float32),
        pltpu.VMEM((1, RPT, 128), jnp.int32),
        pltpu.VMEM((16,), jnp.float32),
        pltpu.VMEM((16, 16), jnp.float32),
        pltpu.VMEM((4, 2, 8, 128), jnp.float32),
        pltpu.VMEM((4, 2, 8, 128), jnp.float32),
        pltpu.VMEM((256, CH), jnp.float32),
        pltpu.VMEM((256, CH), jnp.float32),
        pltpu.VMEM((128,), jnp.float32),
        pltpu.VMEM((256,), jnp.float32),
        pltpu.VMEM((1, 128, CH), jnp.float32),
        pltpu.VMEM((1, 128, CH), jnp.float32),
        pltpu.VMEM_SHARED((V, CH), jnp.float32),
        pltpu.VMEM_SHARED((V,), jnp.float32),
        pltpu.VMEM_SHARED((16, 16), jnp.float32),
        pltpu.VMEM_SHARED((16, 16), jnp.float32),
        pltpu.VMEM_SHARED((16, 16), jnp.float32),
        pltpu.VMEM_SHARED((16, 16), jnp.float32),
        pltpu.SemaphoreType.DMA,
        pltpu.SemaphoreType.DMA,
        pltpu.SemaphoreType.DMA,
        pltpu.SemaphoreType.DMA,
    ],
)
def _vox_sc(ftv, cxv, idxr, vc, vf, xv, yv, zv, iv, sbuf, pv, fb0, fb1, fr0, fr1,
            ones_v, cvb, evb, odb, acc, cnt, px, py, pz, pm,
            fbsem0, fbsem1, scsem0, scsem1):
    c = lax.axis_index("c")
    s = lax.axis_index("s")
    v0 = s * VT
    zf = jnp.zeros((16,), jnp.float32)
    LANE = lax.iota(jnp.int32, 16)
    HALF = jnp.full((16,), 0.5, jnp.float32)
    fbs = (fb0, fb1)
    frs = (fr0, fr1)
    fsems = (fbsem0, fbsem1)
    ssems = (scsem0, scsem1)

    @pl.loop(0, 8)
    def _(i):
        ones_v[pl.ds(i * 16, 16)] = jnp.ones((16,), jnp.float32)

    def colsum(ref):
        t = ref[0]
        for i in range(1, 16):
            t = t + ref[i]
        return t

    def colmax(ref):
        t = ref[0]
        for i in range(1, 16):
            t = jnp.maximum(t, ref[i])
        return t

    def load_chunk(b, j, t):
        return pltpu.async_copy(
            ftv.at[pl.ds(b * 8 + 4 * c, 4), pl.ds(s * RPT + 2 * j, 2), :, :],
            fbs[t], fsems[t])

    @pl.loop(0, B)
    def _(b):
        @pl.loop(0, 256)
        def _(r):
            fr0[r, pl.ds(0, 16)] = zf
            fr0[r, pl.ds(16, 16)] = zf

        @pl.loop(0, 16)
        def _(i):
            cvb[pl.ds(i * 16, 16)] = zf

        for z in range(8):
            pltpu.sync_copy(fr0, acc.at[pl.ds(v0 + z * 256, 256), :])
        for z in range(8):
            pltpu.sync_copy(cvb, cnt.at[pl.ds(v0 + z * 256, 256)])

        pltpu.sync_copy(idxr.at[pl.ds(b, 1), pl.ds(s * RPT, RPT), :], iv)
        pltpu.sync_copy(cxv.at[0, pl.ds(s * RPT, RPT), pl.ds(b, 1), :], xv)
        pltpu.sync_copy(cxv.at[1, pl.ds(s * RPT, RPT), pl.ds(b, 1), :], yv)
        pltpu.sync_copy(cxv.at[2, pl.ds(s * RPT, RPT), pl.ds(b, 1), :], zv)

        @pl.loop(0, RPT, init_carry=(zf, zf, zf))
        def sums(r, carry):
            sx, sy, sz = carry
            for q in range(8):
                d = pl.ds(q * 16, 16)
                sx = sx + xv[r, 0, d]
                sy = sy + yv[r, 0, d]
                sz = sz + zv[r, 0, d]
            return sx, sy, sz

        sx, sy, sz = sums
        for vec, dst in ((sx, px), (sy, py), (sz, pz)):
            sbuf[pl.ds(0, 16)] = vec
            pltpu.sync_copy(sbuf, dst.at[s])
        plsc.subcore_barrier()

        means = []
        for src in (px, py, pz):
            pltpu.sync_copy(src, pv)
            tot = jnp.sum(colsum(pv), axis=0)
            means.append(jnp.broadcast_to(tot * INVN, (16,)))
        mxv, myv, mzv = means

        @pl.loop(0, RPT, init_carry=zf)
        def nmax(r, carry):
            nm = carry
            for q in range(8):
                d = pl.ds(q * 16, 16)
                cx = xv[r, 0, d] - mxv
                cy = yv[r, 0, d] - myv
                cz = zv[r, 0, d] - mzv
                nm = jnp.maximum(nm, cx * cx + cy * cy + cz * cz)
            return nm

        sbuf[pl.ds(0, 16)] = nmax
        pltpu.sync_copy(sbuf, pm.at[s])
        plsc.subcore_barrier()
        pltpu.sync_copy(pm, pv)
        gmax = jnp.broadcast_to(jnp.max(colmax(pv), axis=0), (16,))
        yi = jnp.int32(0x5F3759DF) - (plsc.bitcast(gmax, jnp.int32) >> 1)
        ya = plsc.bitcast(yi, jnp.float32)
        for _ in range(3):
            ya = ya * (1.5 - 0.5 * gmax * ya * ya)
        scalev = 0.5 * ya

        @pl.loop(0, RPT)
        def _(r):
            for q in range(8):
                d = pl.ds(q * 16, 16)
                vx = (xv[r, 0, d] - mxv) * scalev + HALF
                vy = (yv[r, 0, d] - myv) * scalev + HALF
                vz = (zv[r, 0, d] - mzv) * scalev + HALF
                vx = jnp.clip(vx * float(RES), 0.0, float(RES - 1))
                vy = jnp.clip(vy * float(RES), 0.0, float(RES - 1))
                vz = jnp.clip(vz * float(RES), 0.0, float(RES - 1))
                xv[r, 0, d] = vx
                yv[r, 0, d] = vy
                zv[r, 0, d] = vz

        @pl.when(c == 0)
        def _():
            pltpu.sync_copy(xv, vc.at[0, pl.ds(s * RPT, RPT), pl.ds(b, 1), :])
            pltpu.sync_copy(yv, vc.at[1, pl.ds(s * RPT, RPT), pl.ds(b, 1), :])
            pltpu.sync_copy(zv, vc.at[2, pl.ds(s * RPT, RPT), pl.ds(b, 1), :])

        plsc.subcore_barrier()

        load_chunk(b, 0, 0)
        load_chunk(b, 1, 1)

        @pl.loop(0, NCHUNK, step=2)
        def _(j):
            for t in range(2):
                jj = j + t
                fb, fr = fbs[t], frs[t]
                pltpu.make_async_copy(
                    ftv.at[pl.ds(0, 4), pl.ds(0, 2), :, :], fb,
                    fsems[t]).wait()

                @pl.when(jj >= 2)
                def _():
                    pltpu.make_async_copy(
                        fr, acc.at[pl.ds(0, 256), :], ssems[t]).wait()
                    pltpu.make_async_copy(
                        cvb, cnt.at[pl.ds(0, 256)], ssems[t]).wait()

                @pl.loop(0, 64)
                def _(tt):
                    k = tt >> 4
                    pcb = (tt >> 3) & 1
                    r = tt & 7
                    colv = jnp.broadcast_to(8 * k + r, (16,))
                    rowb = jnp.broadcast_to(pcb * 128, (16,)) + LANE
                    for qc in range(8):
                        vals = fb[k, pcb, r, pl.ds(qc * 16, 16)]
                        plsc.store_scatter(fr, [rowb + qc * 16, colv], vals)

                @pl.when(jj + 2 < NCHUNK)
                def _():
                    load_chunk(b, jj + 2, t)

                for q in range(2):
                    row = iv.at[0, 2 * jj + q]
                    pltpu.async_copy(fr.at[pl.ds(q * 128, 128), :],
                                     acc.at[row], ssems[t], add=True)
                    pltpu.async_copy(ones_v, cnt.at[row], ssems[t], add=True)

        for t in range(2):
            pltpu.make_async_copy(frs[t], acc.at[pl.ds(0, 256), :],
                                  ssems[t]).wait()
            pltpu.make_async_copy(cvb, cnt.at[pl.ds(0, 256)], ssems[t]).wait()

        plsc.subcore_barrier()

        @pl.loop(0, 8)
        def _(u):
            vb = v0 + u * 256
            pltpu.sync_copy(acc.at[pl.ds(vb, 256), :], fr0)
            pltpu.sync_copy(cnt.at[pl.ds(vb, 256)], cvb)

            @pl.loop(0, 16)
            def _(g):
                base = g * 16
                inv = 1.0 / jnp.maximum(cvb[pl.ds(base, 16)], 1.0)
                for w in range(16):
                    sw = jnp.sum(jnp.where(LANE == w, inv, 0.0), axis=0)
                    bw = jnp.broadcast_to(sw, (16,))
                    row = base + w
                    half = (base >> 1) + (w >> 1)
                    dst = evb if (w % 2 == 0) else odb
                    dst[0, half, pl.ds(0, 16)] = fr0[row, pl.ds(0, 16)] * bw
                    dst[0, half, pl.ds(16, 16)] = fr0[row, pl.ds(16, 16)] * bw

            vh = vb >> 1
            pltpu.sync_copy(
                evb, vf.at[pl.ds(b, 1), pl.ds(vh, 128), pl.ds(32 * c, 32)])
            pltpu.sync_copy(
                odb, vf.at[pl.ds(b, 1), pl.ds(vh, 128), pl.ds(64 + 32 * c, 32)])


def kernel(coords, features):
    ftv = (features.transpose(0, 2, 1)
           .reshape(4, 8, 8, 512, 128)
           .transpose(0, 1, 3, 2, 4)
           .reshape(32, 512, 8, 128))
    cxv = (coords.transpose(2, 0, 1)
           .reshape(3, 4, 512, 128)
           .transpose(0, 2, 1, 3))
    idx = _idx_tc(coords.transpose(0, 2, 1))
    vc4, vf4 = _vox_sc(ftv, cxv, idx.reshape(B, 512, 128))
    voxel_coords = (vc4.transpose(0, 2, 1, 3)
                    .reshape(3, 4, 65536)
                    .transpose(1, 2, 0))
    voxel_features = vf4.reshape(B, V, C).reshape(B, RES, RES, RES, C)
    return voxel_coords, voxel_features


# speedup vs baseline: 1.0915x; 1.0915x over previous
import functools
import jax
import jax.numpy as jnp
from jax import lax
from jax.experimental import pallas as pl
from jax.experimental.pallas import tpu as pltpu
from jax.experimental.pallas import tpu_sc as plsc

RES = 32
V = RES ** 3
B = 4
N = 65536
C = 64

NC = 2
NS = 16
CH = C // NC
PT = N // NS
VT = V // NS
CHUNK = 512
NCHUNK = PT // CHUNK
QPC = CHUNK // 128
ROWS = N // 128
RPT = ROWS // NS


def _coords_body(ct_ref, vc_ref, idx_ref):
    c3 = ct_ref[0]
    mean = jnp.mean(c3, axis=1, keepdims=True)
    cc = c3 - mean
    n2 = jnp.sum(cc * cc, axis=0, keepdims=True)
    m = jnp.sqrt(jnp.max(n2))
    cn = cc / (m * 2.0) + 0.5
    v = jnp.clip(cn * float(RES), 0.0, float(RES - 1))
    vc_ref[0] = v
    vi = jnp.round(v).astype(jnp.int32)
    idx_ref[0] = vi[0:1] * (RES * RES) + vi[1:2] * RES + vi[2:3]


def _coords_tc(ct):
    return pl.pallas_call(
        _coords_body,
        grid=(B,),
        in_specs=[pl.BlockSpec((1, 3, N), lambda b: (b, 0, 0))],
        out_specs=[
            pl.BlockSpec((1, 3, N), lambda b: (b, 0, 0)),
            pl.BlockSpec((1, 1, N), lambda b: (b, 0, 0)),
        ],
        out_shape=[
            jax.ShapeDtypeStruct((B, 3, N), jnp.float32),
            jax.ShapeDtypeStruct((B, 1, N), jnp.int32),
        ],
    )(ct)


_MESH = plsc.VectorSubcoreMesh(
    core_axis_name="c", subcore_axis_name="s", num_cores=NC, num_subcores=NS)


@functools.partial(
    pl.kernel,
    out_type=[
        jax.ShapeDtypeStruct((B, V, C), jnp.float32),
        jax.ShapeDtypeStruct((B, V), jnp.float32),
    ],
    mesh=_MESH,
    compiler_params=pltpu.CompilerParams(use_tc_tiling_on_sc=False),
    scratch_types=[
        pltpu.VMEM((CHUNK, CH), jnp.float32),
        pltpu.VMEM((RPT, 128), jnp.int32),
        pltpu.VMEM((128,), jnp.float32),
        pltpu.VMEM((CHUNK, CH), jnp.float32),
        pltpu.VMEM((VT,), jnp.float32),
        pltpu.VMEM_SHARED((V, CH), jnp.float32),
        pltpu.VMEM_SHARED((V,), jnp.float32),
    ],
)
def _scatter_sc(feats, idxs, ones_h, z2d, z1d, sums_out, cnts_out,
                fv, iv, ones_v, zv, zc, sums_sp, cnts_sp):
    c = lax.axis_index("c")
    s = lax.axis_index("s")
    ch0 = c * CH
    v0 = s * VT
    pltpu.sync_copy(ones_h, ones_v)
    pltpu.sync_copy(z2d, zv)
    pltpu.sync_copy(z1d, zc)
    for b in range(B):
        for z in range(VT // CHUNK):
            pltpu.sync_copy(zv, sums_sp.at[pl.ds(v0 + z * CHUNK, CHUNK), :])
        pltpu.sync_copy(zc, cnts_sp.at[pl.ds(v0, VT)])
        plsc.subcore_barrier()

        pltpu.sync_copy(idxs.at[b, pl.ds(s * RPT, RPT), :], iv)

        @pl.loop(0, NCHUNK)
        def _chunk(j):
            p0 = s * PT + j * CHUNK
            pltpu.sync_copy(feats.at[b, pl.ds(p0, CHUNK), pl.ds(ch0, CH)], fv)
            for q in range(QPC):
                row = iv.at[j * QPC + q]
                pltpu.sync_copy(fv.at[pl.ds(q * 128, 128), :],
                                sums_sp.at[row], add=True)
                pltpu.sync_copy(ones_v, cnts_sp.at[row], add=True)

        plsc.subcore_barrier()

        pltpu.sync_copy(sums_sp.at[pl.ds(v0, VT), :],
                        sums_out.at[b, pl.ds(v0, VT), pl.ds(ch0, CH)])

        @pl.when(c == 0)
        def _():
            pltpu.sync_copy(cnts_sp.at[pl.ds(v0, VT)],
                            cnts_out.at[b, pl.ds(v0, VT)])


def _div_body(s_ref, c_ref, o_ref):
    cnt = jnp.maximum(c_ref[0], 1.0)
    o_ref[0] = s_ref[0] / cnt


def _div_tc(sums, cnts3):
    return pl.pallas_call(
        _div_body,
        grid=(B, V // VT),
        in_specs=[
            pl.BlockSpec((1, VT, C), lambda b, i: (b, i, 0)),
            pl.BlockSpec((1, VT, 1), lambda b, i: (b, i, 0)),
        ],
        out_specs=pl.BlockSpec((1, VT, C), lambda b, i: (b, i, 0)),
        out_shape=jax.ShapeDtypeStruct((B, V, C), jnp.float32),
    )(sums, cnts3)


def kernel(coords, features):
    ct = coords.transpose(0, 2, 1)
    vc_t, idx = _coords_tc(ct)
    voxel_coords = vc_t.transpose(0, 2, 1)
    idx_rows = idx.reshape(B, ROWS, 128)
    ones_h = jnp.ones((128,), jnp.float32)
    z2d = jnp.zeros((CHUNK, CH), jnp.float32)
    z1d = jnp.zeros((VT,), jnp.float32)
    sums, cnts = _scatter_sc(features, idx_rows, ones_h, z2d, z1d)
    vox = _div_tc(sums, cnts.reshape(B, V, 1))
    voxel_features = vox.reshape(B, RES, RES, RES, C)
    return voxel_coords, voxel_features
